# baseline (speedup 1.0000x reference)
import functools

import jax
import jax.numpy as jnp
from jax import lax
from jax.experimental import pallas as pl
from jax.experimental.pallas import tpu as pltpu
from jax.experimental.pallas import tpu_sc as plsc

TOPK = 100
LANES = 16


@functools.lru_cache(maxsize=None)
def _build_mask_kernel(B, KPAD, NPAD):
    info = plsc.get_sparse_core_info()
    nc = info.num_cores

    mesh = plsc.VectorSubcoreMesh(core_axis_name="c", subcore_axis_name="s")

    @functools.partial(
        pl.kernel,
        out_type=jax.ShapeDtypeStruct((B, NPAD), jnp.float32),
        mesh=mesh,
        scratch_types=[
            pltpu.VMEM((KPAD,), jnp.int32),
            pltpu.VMEM((NPAD,), jnp.float32),
        ],
        compiler_params=pltpu.CompilerParams(needs_layout_passes=False),
    )
    def mask_kernel(pids_hbm, out_hbm, pid_v, mask_v):
        wid = lax.axis_index("s") * nc + lax.axis_index("c")

        @pl.when(wid < B)
        def _():
            pltpu.sync_copy(pids_hbm.at[wid], pid_v)

            def zero_body(i, c):
                mask_v[pl.ds(i * LANES, LANES)] = jnp.zeros(
                    (LANES,), jnp.float32)
                return c

            lax.fori_loop(0, NPAD // LANES, zero_body, 0)

            ones = jnp.ones((LANES,), jnp.float32)

            def scat_body(j, c):
                pv = pid_v[pl.ds(j * LANES, LANES)]
                valid = pv >= 0
                safe = jnp.where(valid, pv, 0)
                plsc.store_scatter(mask_v, [safe], ones, mask=valid)
                return c

            lax.fori_loop(0, KPAD // LANES, scat_body, 0)

            pltpu.sync_copy(mask_v, out_hbm.at[wid])

    return mask_kernel


def _score_body(q_ref, v_ref, m_ref, cs_ref, ci_ref, *, G, D, B, Q, KOUT):
    g = pl.program_id(0)
    neg_inf = jnp.float32(-jnp.inf)

    q = q_ref[...]
    SG = 32
    parts = []
    for c in range(G // SG):
        vc = v_ref[pl.ds(c * SG, SG)]
        s = lax.dot_general(vc.reshape(SG * D, q.shape[1]), q,
                            (((1,), (1,)), ((), ())),
                            preferred_element_type=jnp.float32)
        parts.append(s.reshape(SG, D, B * Q).max(axis=1))
    m = jnp.concatenate(parts, axis=0)
    mt = m.T
    sc = mt.reshape(B, Q, G).sum(axis=1) * (1.0 / Q)
    masked = jnp.where(m_ref[...] > 0, sc, neg_inf)

    lane = lax.broadcasted_iota(jnp.int32, (B, G), 1)
    col = lax.broadcasted_iota(jnp.int32, (B, KOUT), 1)
    base = g * G

    def body(i, carry):
        a_s, a_i, work = carry
        mx = jnp.max(work, axis=1, keepdims=True)
        li = jnp.min(jnp.where(work == mx, lane, G), axis=1,
                     keepdims=True)
        sel = col == i
        a_s = jnp.where(sel, mx, a_s)
        a_i = jnp.where(sel, base + li, a_i)
        work = jnp.where(lane == li, neg_inf, work)
        return a_s, a_i, work

    init = (jnp.full((B, KOUT), neg_inf, jnp.float32),
            jnp.full((B, KOUT), -1, jnp.int32),
            masked)
    a_s, a_i, _ = lax.fori_loop(0, TOPK, body, init)
    cs_ref[...] = a_s
    ci_ref[...] = a_i


@functools.lru_cache(maxsize=None)
def _build_score_call(B, Q, DIM, D, NPAD, G, KOUT):
    NB = NPAD // G
    body = functools.partial(_score_body, G=G, D=D, B=B, Q=Q, KOUT=KOUT)
    return pl.pallas_call(
        body,
        grid=(NB,),
        in_specs=[
            pl.BlockSpec((B * Q, DIM), lambda g: (0, 0)),
            pl.BlockSpec((G, D, DIM), lambda g: (g, 0, 0)),
            pl.BlockSpec((B, G), lambda g: (0, g)),
        ],
        out_specs=(pl.BlockSpec((B, KOUT), lambda g: (0, g)),
                   pl.BlockSpec((B, KOUT), lambda g: (0, g))),
        out_shape=(jax.ShapeDtypeStruct((B, NB * KOUT), jnp.float32),
                   jax.ShapeDtypeStruct((B, NB * KOUT), jnp.int32)),
    )


def _merge_body(cs_ref, ci_ref, os_ref, oi_ref, *, B, W, KOUT):
    neg_inf = jnp.float32(-jnp.inf)
    lane = lax.broadcasted_iota(jnp.int32, (B, W), 1)
    col = lax.broadcasted_iota(jnp.int32, (B, KOUT), 1)
    ids = ci_ref[...]

    def body(i, carry):
        a_s, a_i, work = carry
        mx = jnp.max(work, axis=1, keepdims=True)
        li = jnp.min(jnp.where(work == mx, lane, W), axis=1,
                     keepdims=True)
        first = lane == li
        pid = jnp.min(jnp.where(first, ids, jnp.int32(2**30)),
                      axis=1, keepdims=True)
        sel = col == i
        a_s = jnp.where(sel, mx, a_s)
        a_i = jnp.where(sel, pid, a_i)
        work = jnp.where(first, neg_inf, work)
        return a_s, a_i, work

    init = (jnp.full((B, KOUT), neg_inf, jnp.float32),
            jnp.full((B, KOUT), -1, jnp.int32),
            cs_ref[...])
    a_s, a_i, _ = lax.fori_loop(0, TOPK, body, init)
    os_ref[...] = a_s
    oi_ref[...] = a_i


@functools.lru_cache(maxsize=None)
def _build_merge_call(B, W, KOUT):
    body = functools.partial(_merge_body, B=B, W=W, KOUT=KOUT)
    return pl.pallas_call(
        body,
        out_shape=(jax.ShapeDtypeStruct((B, KOUT), jnp.float32),
                   jax.ShapeDtypeStruct((B, KOUT), jnp.int32)),
    )


def kernel(q_vectors, pids, k, vectors, boundaries):
    B, Q, DIM = q_vectors.shape
    N, D, _ = vectors.shape
    K = pids.shape[1]

    G = 640
    NPAD = ((N + G - 1) // G) * G
    KPAD = ((K + 127) // 128) * 128

    p = pids - boundaries[0]
    p = jnp.where((p < 0) | (p >= N), -1, p)
    p_pad = jnp.pad(p, ((0, 0), (0, KPAD - K)), constant_values=-1)

    mask = _build_mask_kernel(B, KPAD, NPAD)(p_pad)

    q2 = q_vectors.reshape(B * Q, DIM)
    KOUT = ((TOPK + 127) // 128) * 128
    cand_s, cand_i = _build_score_call(B, Q, DIM, D, NPAD, G, KOUT)(
        q2, vectors, mask)

    NB = NPAD // G
    s_pad, i_pad = _build_merge_call(B, NB * KOUT, KOUT)(cand_s, cand_i)

    scores = s_pad[:, :TOPK] + k * 0
    upids = i_pad[:, :TOPK]
    return scores, upids


# speedup vs baseline: 3.2107x; 3.2107x over previous
import functools

import jax
import jax.numpy as jnp
from jax import lax
from jax.experimental import pallas as pl
from jax.experimental.pallas import tpu as pltpu
from jax.experimental.pallas import tpu_sc as plsc

TOPK = 100
LANES = 16


@functools.lru_cache(maxsize=None)
def _build_mask_kernel(B, KPAD, NPAD):
    info = plsc.get_sparse_core_info()
    nc = info.num_cores

    mesh = plsc.VectorSubcoreMesh(core_axis_name="c", subcore_axis_name="s")

    @functools.partial(
        pl.kernel,
        out_type=jax.ShapeDtypeStruct((B, NPAD), jnp.float32),
        mesh=mesh,
        scratch_types=[
            pltpu.VMEM((KPAD,), jnp.int32),
            pltpu.VMEM((NPAD,), jnp.float32),
        ],
        compiler_params=pltpu.CompilerParams(needs_layout_passes=False),
    )
    def mask_kernel(pids_hbm, out_hbm, pid_v, mask_v):
        wid = lax.axis_index("s") * nc + lax.axis_index("c")

        @pl.when(wid < B)
        def _():
            pltpu.sync_copy(pids_hbm.at[wid], pid_v)

            def zero_body(i, c):
                mask_v[pl.ds(i * LANES, LANES)] = jnp.zeros(
                    (LANES,), jnp.float32)
                return c

            lax.fori_loop(0, NPAD // LANES, zero_body, 0)

            ones = jnp.ones((LANES,), jnp.float32)

            def scat_body(j, c):
                pv = pid_v[pl.ds(j * LANES, LANES)]
                valid = pv >= 0
                safe = jnp.where(valid, pv, 0)
                plsc.store_scatter(mask_v, [safe], ones, mask=valid)
                return c

            lax.fori_loop(0, KPAD // LANES, scat_body, 0)

            pltpu.sync_copy(mask_v, out_hbm.at[wid])

    return mask_kernel


def _fused_body(q_ref, v_ref, m_ref, os_ref, oi_ref, sc_ref,
                *, G, D, B, Q, NPAD, KOUT, NB):
    g = pl.program_id(0)
    neg_inf = jnp.float32(-jnp.inf)

    q = q_ref[...]
    SG = 32
    parts = []
    for c in range(G // SG):
        vc = v_ref[pl.ds(c * SG, SG)]
        s = lax.dot_general(vc.reshape(SG * D, q.shape[1]), q,
                            (((1,), (1,)), ((), ())),
                            preferred_element_type=jnp.float32)
        parts.append(s.reshape(SG, D, B * Q).max(axis=1))
    m = jnp.concatenate(parts, axis=0)
    mt = m.T
    sc = mt.reshape(B, Q, G).sum(axis=1) * (1.0 / Q)
    sc_ref[:, pl.ds(g * G, G)] = jnp.where(m_ref[...] > 0, sc, neg_inf)

    @pl.when(g == NB - 1)
    def _():
        iota = lax.broadcasted_iota(jnp.int32, (B, NPAD), 1)
        col_iota = lax.broadcasted_iota(jnp.int32, (B, KOUT), 1)

        def body(i, carry):
            acc_s, acc_i = carry
            sall = sc_ref[...]
            mx = jnp.max(sall, axis=1, keepdims=True)
            hit = sall == mx
            idx = jnp.max(jnp.where(hit, iota, -1), axis=1,
                          keepdims=True)
            col = col_iota == i
            acc_s = jnp.where(col, mx, acc_s)
            acc_i = jnp.where(col, idx, acc_i)
            sc_ref[...] = jnp.where(iota == idx, neg_inf, sall)
            return acc_s, acc_i

        init = (jnp.full((B, KOUT), neg_inf, jnp.float32),
                jnp.full((B, KOUT), -1, jnp.int32))
        acc_s, acc_i = lax.fori_loop(0, TOPK, body, init, unroll=5)
        os_ref[...] = acc_s
        oi_ref[...] = acc_i


@functools.lru_cache(maxsize=None)
def _build_fused_call(B, Q, DIM, N, D, NPAD, G, KOUT):
    NB = NPAD // G
    body = functools.partial(_fused_body, G=G, D=D, B=B, Q=Q,
                             NPAD=NPAD, KOUT=KOUT, NB=NB)
    return pl.pallas_call(
        body,
        grid=(NB,),
        in_specs=[
            pl.BlockSpec((B * Q, DIM), lambda g: (0, 0)),
            pl.BlockSpec((G, D, DIM), lambda g: (g, 0, 0)),
            pl.BlockSpec((B, G), lambda g: (0, g)),
        ],
        out_specs=(pl.BlockSpec((B, KOUT), lambda g: (0, 0)),
                   pl.BlockSpec((B, KOUT), lambda g: (0, 0))),
        out_shape=(jax.ShapeDtypeStruct((B, KOUT), jnp.float32),
                   jax.ShapeDtypeStruct((B, KOUT), jnp.int32)),
        scratch_shapes=[pltpu.VMEM((B, NPAD), jnp.float32)],
    )


def kernel(q_vectors, pids, k, vectors, boundaries):
    B, Q, DIM = q_vectors.shape
    N, D, _ = vectors.shape
    K = pids.shape[1]

    G = 640
    NPAD = ((N + G - 1) // G) * G
    KPAD = ((K + 127) // 128) * 128

    p = pids - boundaries[0]
    p = jnp.where((p < 0) | (p >= N), -1, p)
    p_pad = jnp.pad(p, ((0, 0), (0, KPAD - K)), constant_values=-1)

    mask = _build_mask_kernel(B, KPAD, NPAD)(p_pad)

    q2 = q_vectors.reshape(B * Q, DIM)
    KOUT = ((TOPK + 127) // 128) * 128
    s_pad, i_pad = _build_fused_call(B, Q, DIM, N, D, NPAD, G, KOUT)(
        q2, vectors, mask)

    scores = s_pad[:, :TOPK] + k * 0
    upids = i_pad[:, :TOPK]
    return scores, upids


# speedup vs baseline: 3.2463x; 1.0111x over previous
import functools

import jax
import jax.numpy as jnp
from jax import lax
from jax.experimental import pallas as pl
from jax.experimental.pallas import tpu as pltpu
from jax.experimental.pallas import tpu_sc as plsc

TOPK = 100
LANES = 16


@functools.lru_cache(maxsize=None)
def _build_mask_kernel(B, KPAD, NPAD):
    info = plsc.get_sparse_core_info()
    nc = info.num_cores

    mesh = plsc.VectorSubcoreMesh(core_axis_name="c", subcore_axis_name="s")

    @functools.partial(
        pl.kernel,
        out_type=jax.ShapeDtypeStruct((B, NPAD), jnp.float32),
        mesh=mesh,
        scratch_types=[
            pltpu.VMEM((KPAD,), jnp.int32),
            pltpu.VMEM((NPAD,), jnp.float32),
        ],
        compiler_params=pltpu.CompilerParams(needs_layout_passes=False),
    )
    def mask_kernel(pids_hbm, zeros_hbm, out_hbm, pid_v, mask_v):
        wid = lax.axis_index("s") * nc + lax.axis_index("c")

        @pl.when(wid < B)
        def _():
            pltpu.sync_copy(pids_hbm.at[wid], pid_v)
            pltpu.sync_copy(zeros_hbm, mask_v)

            ones = jnp.ones((LANES,), jnp.float32)

            def scat_body(j, c):
                pv = pid_v[pl.ds(j * LANES, LANES)]
                valid = pv >= 0
                safe = jnp.where(valid, pv, 0)
                plsc.store_scatter(mask_v, [safe], ones, mask=valid)
                return c

            lax.fori_loop(0, KPAD // LANES, scat_body, 0, unroll=4)

            pltpu.sync_copy(mask_v, out_hbm.at[wid])

    return mask_kernel


def _fused_body(q_ref, v_ref, m_ref, os_ref, oi_ref, sc_ref,
                *, G, D, B, Q, NPAD, KOUT, NB):
    g = pl.program_id(0)
    neg_inf = jnp.float32(-jnp.inf)

    q = q_ref[...]
    SG = 32
    parts = []
    for c in range(G // SG):
        vc = v_ref[pl.ds(c * SG, SG)]
        s = lax.dot_general(vc.reshape(SG * D, q.shape[1]), q,
                            (((1,), (1,)), ((), ())),
                            preferred_element_type=jnp.float32)
        parts.append(s.reshape(SG, D, B * Q).max(axis=1))
    m = jnp.concatenate(parts, axis=0)
    mt = m.T
    sc = mt.reshape(B, Q, G).sum(axis=1) * (1.0 / Q)
    sc_ref[:, pl.ds(g * G, G)] = jnp.where(m_ref[...] > 0, sc, neg_inf)

    @pl.when(g == NB - 1)
    def _():
        iota = lax.broadcasted_iota(jnp.int32, (B, NPAD), 1)
        col_iota = lax.broadcasted_iota(jnp.int32, (B, KOUT), 1)

        def body(i, carry):
            acc_s, acc_i = carry
            sall = sc_ref[...]
            mx = jnp.max(sall, axis=1, keepdims=True)
            hit = sall == mx
            idx = jnp.max(jnp.where(hit, iota, -1), axis=1,
                          keepdims=True)
            col = col_iota == i
            acc_s = jnp.where(col, mx, acc_s)
            acc_i = jnp.where(col, idx, acc_i)
            sc_ref[...] = jnp.where(iota == idx, neg_inf, sall)
            return acc_s, acc_i

        init = (jnp.full((B, KOUT), neg_inf, jnp.float32),
                jnp.full((B, KOUT), -1, jnp.int32))
        acc_s, acc_i = lax.fori_loop(0, TOPK, body, init, unroll=10)
        os_ref[...] = acc_s
        oi_ref[...] = acc_i


@functools.lru_cache(maxsize=None)
def _build_fused_call(B, Q, DIM, N, D, NPAD, G, KOUT):
    NB = NPAD // G
    body = functools.partial(_fused_body, G=G, D=D, B=B, Q=Q,
                             NPAD=NPAD, KOUT=KOUT, NB=NB)
    return pl.pallas_call(
        body,
        grid=(NB,),
        in_specs=[
            pl.BlockSpec((B * Q, DIM), lambda g: (0, 0)),
            pl.BlockSpec((G, D, DIM), lambda g: (g, 0, 0)),
            pl.BlockSpec((B, G), lambda g: (0, g)),
        ],
        out_specs=(pl.BlockSpec((B, KOUT), lambda g: (0, 0)),
                   pl.BlockSpec((B, KOUT), lambda g: (0, 0))),
        out_shape=(jax.ShapeDtypeStruct((B, KOUT), jnp.float32),
                   jax.ShapeDtypeStruct((B, KOUT), jnp.int32)),
        scratch_shapes=[pltpu.VMEM((B, NPAD), jnp.float32)],
    )


def kernel(q_vectors, pids, k, vectors, boundaries):
    B, Q, DIM = q_vectors.shape
    N, D, _ = vectors.shape
    K = pids.shape[1]

    G = 640
    NPAD = ((N + G - 1) // G) * G
    KPAD = ((K + 127) // 128) * 128

    p = pids - boundaries[0]
    p = jnp.where((p < 0) | (p >= N), -1, p)
    p_pad = jnp.pad(p, ((0, 0), (0, KPAD - K)), constant_values=-1)

    mask = _build_mask_kernel(B, KPAD, NPAD)(
        p_pad, jnp.zeros((NPAD,), jnp.float32))

    q2 = q_vectors.reshape(B * Q, DIM)
    KOUT = ((TOPK + 127) // 128) * 128
    s_pad, i_pad = _build_fused_call(B, Q, DIM, N, D, NPAD, G, KOUT)(
        q2, vectors, mask)

    scores = s_pad[:, :TOPK] + k * 0
    upids = i_pad[:, :TOPK]
    return scores, upids
